# baseline (speedup 1.0000x reference)
import functools

import jax
import jax.numpy as jnp
from jax import lax
from jax.experimental import pallas as pl
from jax.experimental.pallas import tpu as pltpu
from jax.experimental.pallas import tpu_sc as plsc

_NC = 2
_NS = 16
_NW = _NC * _NS
_BK = 128
_GD = 5
_AH = 3
_PER = 20
_NSPL = 2


@functools.partial(jax.jit, static_argnames=("seq",))
def _gather(ids_t, tbl_p, seq):
    batch = ids_t.shape[1]
    padw = tbl_p.shape[1]
    nsb = seq // 8
    hw = _BK // _NSPL
    mesh = plsc.VectorSubcoreMesh(core_axis_name="c", subcore_axis_name="s")

    @functools.partial(
        pl.kernel,
        out_type=jax.ShapeDtypeStruct((seq, batch, padw), jnp.float32),
        mesh=mesh,
        compiler_params=pltpu.CompilerParams(use_tc_tiling_on_sc=True,
                                             needs_layout_passes=False),
        scratch_types=[
            pltpu.VMEM((2, 8, _BK), jnp.int32),
            pltpu.VMEM((_GD, _BK, padw), jnp.float32),
            pltpu.SemaphoreType.DMA((_GD,)),
            pltpu.SemaphoreType.DMA((_GD,)),
            pltpu.SemaphoreType.DMA,
        ],
    )
    def k(ids_hbm, tbl_hbm, out_hbm, idx_v, g_v, gsem, wsem, isem):
        wid = lax.axis_index("s") * _NC + lax.axis_index("c")
        b0 = wid * _BK

        pltpu.sync_copy(ids_hbm.at[pl.ds(0, 8), pl.ds(b0, _BK)], idx_v.at[0])

        def launch_gather(t, slot):
            sb = t // 8
            r = lax.rem(t, 8)
            for h in range(_NSPL):
                pltpu.async_copy(
                    tbl_hbm.at[idx_v.at[lax.rem(sb, 2), r, pl.ds(h * hw, hw)]],
                    g_v.at[slot, pl.ds(h * hw, hw)], gsem.at[slot])

        def block(Gi, carry):
            t0 = Gi * _PER
            for u in range(_PER):
                t = t0 + u
                sb = t // 8
                r = lax.rem(t, 8)
                sl = u % _GD
                nsl = (u + _AH) % _GD

                pltpu.make_async_copy(
                    tbl_hbm.at[pl.ds(0, _BK)], g_v.at[sl], gsem.at[sl]).wait()

                @pl.when(jnp.logical_and(r == 0, sb + 1 < nsb))
                def _():
                    pltpu.async_copy(
                        ids_hbm.at[pl.ds((sb + 1) * 8, 8), pl.ds(b0, _BK)],
                        idx_v.at[lax.rem(sb + 1, 2)], isem)

                pltpu.async_copy(
                    g_v.at[sl], out_hbm.at[t, pl.ds(b0, _BK)], wsem.at[sl])

                @pl.when(jnp.logical_and(r == 4, sb + 1 < nsb))
                def _():
                    pltpu.make_async_copy(
                        ids_hbm.at[pl.ds(0, 8), pl.ds(0, _BK)], idx_v.at[0],
                        isem).wait()

                @pl.when(t + _AH < seq)
                def _():
                    @pl.when(t >= 2)
                    def _():
                        pltpu.make_async_copy(
                            tbl_hbm.at[pl.ds(0, _BK)], g_v.at[nsl],
                            wsem.at[nsl]).wait()

                    launch_gather(t + _AH, nsl)

            return carry

        for slot in range(_AH):
            launch_gather(slot, slot)
        lax.fori_loop(0, seq // _PER, block, 0)

        for slot in range(_GD):
            pltpu.make_async_copy(
                tbl_hbm.at[pl.ds(0, _BK)], g_v.at[slot], wsem.at[slot]).wait()

    return k(ids_t, tbl_p)


def kernel(input_ids, token_table, position_table):
    b, s = input_ids.shape
    v, dim = token_table.shape
    assert b == _NW * _BK and s % _PER == 0 and dim == 64
    ids_t = input_ids.T.astype(jnp.int32)
    tbl_p = token_table @ jnp.eye(dim, 128, dtype=jnp.float32)
    g = _gather(ids_t, tbl_p, s)
    out_t = g[:, :, :dim].transpose(0, 2, 1) + position_table[:s][:, :, None]
    return out_t.transpose(2, 0, 1)
